# baseline (speedup 1.0000x reference)
import functools
import math

import jax
import jax.numpy as jnp
from jax import lax
from jax.experimental import pallas as pl
from jax.experimental.pallas import tpu as pltpu
from jax.experimental.pallas import tpu_sc as plsc

_NC = 2
_NS = 16
_NW = _NC * _NS
_LANES = 16
_CHUNK = 512


def _make_embed(n_rows: int, vocab: int, d: int):
    assert n_rows % _NW == 0
    rows_per_w = n_rows // _NW
    assert rows_per_w % _CHUNK == 0
    n_chunks = rows_per_w // _CHUNK
    scale = jnp.float32(math.sqrt(d))
    mesh = plsc.VectorSubcoreMesh(core_axis_name="c", subcore_axis_name="s")

    @functools.partial(
        pl.kernel,
        mesh=mesh,
        out_type=jax.ShapeDtypeStruct((n_rows, d), jnp.float32),
        scratch_types=[
            pltpu.VMEM((_CHUNK,), jnp.int32),
            pltpu.VMEM((_CHUNK, d), jnp.float32),
            pltpu.SemaphoreType.DMA,
        ],
        compiler_params=pltpu.CompilerParams(use_tc_tiling_on_sc=False),
    )
    def embed(idx_hbm, table_hbm, out_hbm, idx_v, rows_v, sem):
        wid = lax.axis_index("s") * _NC + lax.axis_index("c")
        base = wid * rows_per_w

        def chunk_body(c, carry):
            off = base + c * _CHUNK
            pltpu.sync_copy(idx_hbm.at[pl.ds(off, _CHUNK)], idx_v)
            pltpu.async_copy(table_hbm.at[idx_v], rows_v, sem).wait()

            def row_body(r, carry2):
                for j in range(d // _LANES):
                    sl = pl.ds(j * _LANES, _LANES)
                    rows_v[r, sl] = rows_v[r, sl] * scale
                return carry2

            lax.fori_loop(0, _CHUNK, row_body, 0, unroll=2)
            pltpu.sync_copy(rows_v, out_hbm.at[pl.ds(off, _CHUNK)])
            return carry

        lax.fori_loop(0, n_chunks, chunk_body, 0)

    return embed


def kernel(x, table):
    b, s = x.shape
    vocab, d = table.shape
    n_rows = b * s
    embed = _make_embed(n_rows, vocab, d)
    out = embed(x.reshape(n_rows), table)
    return out.reshape(b, s, d)


# speedup vs baseline: 1.0865x; 1.0865x over previous
import functools
import math

import jax
import jax.numpy as jnp
from jax import lax
from jax.experimental import pallas as pl
from jax.experimental.pallas import tpu as pltpu
from jax.experimental.pallas import tpu_sc as plsc

_NC = 2
_NS = 16
_NW = _NC * _NS
_LANES = 16
_CHUNK = 640


def _make_embed(n_rows: int, vocab: int, d: int):
    assert n_rows % _NW == 0
    rows_per_w = n_rows // _NW
    assert rows_per_w % (2 * _CHUNK) == 0
    n_pairs = rows_per_w // (2 * _CHUNK)
    scale = jnp.float32(math.sqrt(d))
    mesh = plsc.VectorSubcoreMesh(core_axis_name="c", subcore_axis_name="s")

    @functools.partial(
        pl.kernel,
        mesh=mesh,
        out_type=jax.ShapeDtypeStruct((n_rows, d), jnp.float32),
        scratch_types=[
            pltpu.VMEM((rows_per_w,), jnp.int32),
            pltpu.VMEM((_CHUNK, d), jnp.float32),
            pltpu.VMEM((_CHUNK, d), jnp.float32),
            pltpu.SemaphoreType.DMA,
            pltpu.SemaphoreType.DMA,
        ],
        compiler_params=pltpu.CompilerParams(use_tc_tiling_on_sc=False),
    )
    def embed(idx_hbm, table_hbm, out_hbm, idx_v, rows0, rows1, sem0, sem1):
        wid = lax.axis_index("s") * _NC + lax.axis_index("c")
        base = wid * rows_per_w
        pltpu.sync_copy(idx_hbm.at[pl.ds(base, rows_per_w)], idx_v)

        def start_gather(c, rows_v, sem):
            pltpu.async_copy(
                table_hbm.at[idx_v.at[pl.ds(c * _CHUNK, _CHUNK)]], rows_v, sem
            )

        def finish_chunk(c, rows_v, sem):
            pltpu.make_async_copy(
                table_hbm.at[idx_v.at[pl.ds(c * _CHUNK, _CHUNK)]], rows_v, sem
            ).wait()

            @plsc.parallel_loop(0, _CHUNK, step=1, unroll=4)
            def scale_body(r):
                for j in range(d // _LANES):
                    sl = pl.ds(j * _LANES, _LANES)
                    rows_v[r, sl] = rows_v[r, sl] * scale

            pltpu.sync_copy(rows_v, out_hbm.at[pl.ds(base + c * _CHUNK, _CHUNK)])

        start_gather(0, rows0, sem0)

        def pair_body(p, carry):
            c = 2 * p
            start_gather(c + 1, rows1, sem1)
            finish_chunk(c, rows0, sem0)

            @pl.when(p + 1 < n_pairs)
            def _():
                start_gather(c + 2, rows0, sem0)

            finish_chunk(c + 1, rows1, sem1)
            return carry

        lax.fori_loop(0, n_pairs, pair_body, 0)

    return embed


def kernel(x, table):
    b, s = x.shape
    vocab, d = table.shape
    n_rows = b * s
    embed = _make_embed(n_rows, vocab, d)
    out = embed(x.reshape(n_rows), table)
    return out.reshape(b, s, d)
